# baseline (speedup 1.0000x reference)
import functools

import jax
import jax.numpy as jnp
from jax import lax
from jax.experimental import pallas as pl
from jax.experimental.pallas import tpu as pltpu
from jax.experimental.pallas import tpu_sc as plsc

DIM = 128
CHUNK = 128
NUM_CORES = 2
NUM_SUBCORES = 16
NW = NUM_CORES * NUM_SUBCORES


def kernel(inputs, table):
    batch, seq = inputs.shape
    total = batch * seq
    per_w = total // NW
    n_chunks = per_w // CHUNK
    idx = inputs.T.reshape(total).astype(jnp.int32)

    mesh = plsc.VectorSubcoreMesh(core_axis_name="c", subcore_axis_name="s")

    @functools.partial(
        pl.kernel,
        mesh=mesh,
        out_type=jax.ShapeDtypeStruct((total, DIM), jnp.float32),
        scratch_types=[
            pltpu.VMEM((per_w,), jnp.int32),
            pltpu.VMEM((CHUNK, DIM), jnp.float32),
            pltpu.VMEM((CHUNK, DIM), jnp.float32),
            pltpu.SemaphoreType.DMA,
            pltpu.SemaphoreType.DMA,
            pltpu.SemaphoreType.DMA,
            pltpu.SemaphoreType.DMA,
        ],
    )
    def run(idx_hbm, table_hbm, out_hbm, idx_v, buf_a, buf_b,
            gsem_a, gsem_b, osem_a, osem_b):
        wid = lax.axis_index("s") * NUM_CORES + lax.axis_index("c")
        base = wid * per_w
        pltpu.sync_copy(idx_hbm.at[pl.ds(base, per_w)], idx_v)

        def g_src(j):
            return table_hbm.at[idx_v.at[pl.ds(j * CHUNK, CHUNK)]]

        def o_dst(j):
            return out_hbm.at[pl.ds(base + j * CHUNK, CHUNK)]

        pltpu.async_copy(g_src(0), buf_a, gsem_a)
        pltpu.async_copy(g_src(1), buf_b, gsem_b)

        def body(i, carry):
            j0 = 2 * i
            j1 = j0 + 1
            pltpu.make_async_copy(g_src(j0), buf_a, gsem_a).wait()
            pltpu.async_copy(buf_a, o_dst(j0), osem_a)
            pltpu.make_async_copy(g_src(j1), buf_b, gsem_b).wait()
            pltpu.async_copy(buf_b, o_dst(j1), osem_b)
            pltpu.make_async_copy(buf_a, o_dst(j0), osem_a).wait()

            @pl.when(j0 + 2 < n_chunks)
            def _():
                pltpu.async_copy(g_src(j0 + 2), buf_a, gsem_a)

            pltpu.make_async_copy(buf_b, o_dst(j1), osem_b).wait()

            @pl.when(j1 + 2 < n_chunks)
            def _():
                pltpu.async_copy(g_src(j1 + 2), buf_b, gsem_b)

            return carry

        lax.fori_loop(0, n_chunks // 2, body, 0)

    out = run(idx, table)
    return out.reshape(seq, batch, DIM).transpose(1, 0, 2)


# speedup vs baseline: 1.0367x; 1.0367x over previous
import functools

import jax
import jax.numpy as jnp
from jax import lax
from jax.experimental import pallas as pl
from jax.experimental.pallas import tpu as pltpu
from jax.experimental.pallas import tpu_sc as plsc

DIM = 128
CHUNK = 256
NUM_CORES = 2
NUM_SUBCORES = 16
NW = NUM_CORES * NUM_SUBCORES


def kernel(inputs, table):
    batch, seq = inputs.shape
    total = batch * seq
    per_w = total // NW
    n_chunks = per_w // CHUNK
    idx = inputs.T.reshape(total).astype(jnp.int32)

    mesh = plsc.VectorSubcoreMesh(core_axis_name="c", subcore_axis_name="s")

    @functools.partial(
        pl.kernel,
        mesh=mesh,
        out_type=jax.ShapeDtypeStruct((total, DIM), jnp.float32),
        scratch_types=[
            pltpu.VMEM((per_w,), jnp.int32),
            pltpu.VMEM((CHUNK, DIM), jnp.float32),
            pltpu.VMEM((CHUNK, DIM), jnp.float32),
            pltpu.SemaphoreType.DMA,
            pltpu.SemaphoreType.DMA,
            pltpu.SemaphoreType.DMA,
            pltpu.SemaphoreType.DMA,
        ],
    )
    def run(idx_hbm, table_hbm, out_hbm, idx_v, buf_a, buf_b,
            gsem_a, gsem_b, osem_a, osem_b):
        wid = lax.axis_index("s") * NUM_CORES + lax.axis_index("c")
        base = wid * per_w
        pltpu.sync_copy(idx_hbm.at[pl.ds(base, per_w)], idx_v)

        def g_src(j):
            return table_hbm.at[idx_v.at[pl.ds(j * CHUNK, CHUNK)]]

        def o_dst(j):
            return out_hbm.at[pl.ds(base + j * CHUNK, CHUNK)]

        pltpu.async_copy(g_src(0), buf_a, gsem_a)
        pltpu.async_copy(g_src(1), buf_b, gsem_b)

        def body(i, carry):
            j0 = 2 * i
            j1 = j0 + 1
            pltpu.make_async_copy(g_src(j0), buf_a, gsem_a).wait()
            pltpu.async_copy(buf_a, o_dst(j0), osem_a)
            pltpu.make_async_copy(g_src(j1), buf_b, gsem_b).wait()
            pltpu.async_copy(buf_b, o_dst(j1), osem_b)
            pltpu.make_async_copy(buf_a, o_dst(j0), osem_a).wait()
            pltpu.async_copy(g_src(j0 + 2), buf_a, gsem_a)
            pltpu.make_async_copy(buf_b, o_dst(j1), osem_b).wait()

            @pl.when(j1 + 2 < n_chunks)
            def _():
                pltpu.async_copy(g_src(j1 + 2), buf_b, gsem_b)

            return carry

        lax.fori_loop(0, (n_chunks - 1) // 2, body, 0)

        pltpu.make_async_copy(g_src(n_chunks - 1), buf_a, gsem_a).wait()
        pltpu.sync_copy(buf_a, o_dst(n_chunks - 1))

    out = run(idx, table)
    return out.reshape(seq, batch, DIM).transpose(1, 0, 2)
